# baseline (speedup 1.0000x reference)
import functools

import jax
import jax.numpy as jnp
from jax.experimental import pallas as pl
from jax.experimental.pallas import tpu as pltpu

_EPS = 1e-5


def _fused_kernel(x_ref, w1_ref, b1_ref, w2_ref, b2_ref, ws_ref, bs_ref,
                  gs_ref, betas_ref, wf_ref, bf_ref, gf_ref, bf2_ref,
                  out_ref,
                  csum_ref, zall_ref, wcat_ref, bcat_ref, gcat_ref,
                  betacat_ref, wfc_ref, bfc_ref,
                  *, n_rows, n_blocks, n_strategies, d):
    t = pl.program_id(0)
    i = pl.program_id(1)
    blk = x_ref.shape[0]

    @pl.when(t == 0)
    def _pass0():
        n_int = x_ref.shape[0]
        n_chunks = 8
        chunk = n_int // n_chunks
        parts = [jnp.sum(x_ref[k * chunk:(k + 1) * chunk, :], axis=0,
                         keepdims=True)
                 for k in range(n_chunks)]
        if n_int % n_chunks:
            parts.append(jnp.sum(x_ref[n_chunks * chunk:, :], axis=0,
                                 keepdims=True))
        while len(parts) > 1:
            parts = [parts[j] + parts[j + 1]
                     for j in range(0, len(parts) - 1, 2)
                     ] + ([parts[-1]] if len(parts) % 2 else [])
        part = parts[0]

        @pl.when(i == 0)
        def _():
            csum_ref[...] = part
            for s in range(n_strategies):
                sl = slice(s * d, (s + 1) * d)
                w = ws_ref[s]
                wcat_ref[:, sl] = w - jnp.mean(w, axis=1, keepdims=True)
                b = bs_ref[s:s + 1, :]
                bcat_ref[0:1, sl] = b - jnp.mean(b)
            wf = wf_ref[...]
            wfc_ref[...] = wf - jnp.mean(wf, axis=1, keepdims=True)
            bfc_ref[...] = bf_ref[...] - jnp.mean(bf_ref[...])

        @pl.when(i > 0)
        def _():
            csum_ref[...] = csum_ref[...] + part

        h = (jnp.dot(x_ref[...], wcat_ref[...],
                     preferred_element_type=jnp.float32) + bcat_ref[...])
        inv_d = 1.0 / d
        zs = []
        for s in range(n_strategies):
            hs = h[:, s * d:(s + 1) * d]
            ss = jnp.sum(hs * hs, axis=-1, keepdims=True) * inv_d
            zs.append(hs * jax.lax.rsqrt(ss + _EPS))
        zall_ref[pl.ds(i * blk, blk), :] = jnp.concatenate(
            zs, axis=1).astype(jnp.bfloat16)

        @pl.when(i == n_blocks - 1)
        def _():
            gr = csum_ref[...] * (1.0 / n_rows)
            h = jnp.maximum(
                jnp.dot(gr, w1_ref[...], preferred_element_type=jnp.float32)
                + b1_ref[...], 0.0)
            logits = (jnp.dot(h, w2_ref[...],
                              preferred_element_type=jnp.float32)
                      + b2_ref[...])
            m = jnp.max(logits, axis=-1, keepdims=True)
            e = jnp.exp(logits - m)
            sm = e / jnp.sum(e, axis=-1, keepdims=True)
            for s in range(n_strategies):
                sl = slice(s * d, (s + 1) * d)
                sw_s = sm[0:1, s:s + 1]
                gcat_ref[0:1, sl] = gs_ref[s:s + 1, :] * sw_s
                betacat_ref[0:1, sl] = betas_ref[s:s + 1, :] * sw_s

    @pl.when(t == 1)
    def _pass1():
        zb = zall_ref[pl.ds(i * blk, blk), :].astype(jnp.float32)
        acc = jnp.zeros((blk, d), jnp.float32)
        for s in range(n_strategies):
            sl = slice(s * d, (s + 1) * d)
            acc = acc + jnp.maximum(
                zb[:, sl] * gcat_ref[0:1, sl] + betacat_ref[0:1, sl], 0.0)
        y = (jnp.dot(acc, wfc_ref[...], preferred_element_type=jnp.float32)
             + bfc_ref[...])
        ss = jnp.sum(y * y, axis=-1, keepdims=True) * (1.0 / d)
        r = jax.lax.rsqrt(ss + _EPS)
        out_ref[...] = jnp.maximum(y * r * gf_ref[...] + bf2_ref[...], 0.0)


def kernel(x, edge_index, W1, b1, W2, b2, Ws, bs, gs, betas, Wf, bf, gf, bf2):
    del edge_index
    n, d = x.shape
    s = Ws.shape[0]
    dh = W1.shape[1]

    blk = 5000
    assert n % blk == 0
    nb = n // blk
    const = lambda t, i: (0, 0)
    f32 = jnp.float32

    out = pl.pallas_call(
        functools.partial(_fused_kernel, n_rows=float(n), n_blocks=nb,
                          n_strategies=s, d=d),
        grid=(2, nb),
        in_specs=[
            pl.BlockSpec((blk, d),
                         lambda t, i: (jnp.where(t == 0, i, nb - 1), 0)),
            pl.BlockSpec((d, dh), const),
            pl.BlockSpec((1, dh), const),
            pl.BlockSpec((dh, s), const),
            pl.BlockSpec((1, s), const),
            pl.BlockSpec((s, d, d), lambda t, i: (0, 0, 0)),
            pl.BlockSpec((s, d), const),
            pl.BlockSpec((s, d), const),
            pl.BlockSpec((s, d), const),
            pl.BlockSpec((d, d), const),
            pl.BlockSpec((1, d), const),
            pl.BlockSpec((1, d), const),
            pl.BlockSpec((1, d), const),
        ],
        out_specs=pl.BlockSpec((blk, d),
                               lambda t, i: (jnp.where(t == 0, 0, i), 0)),
        out_shape=jax.ShapeDtypeStruct((n, d), x.dtype),
        scratch_shapes=[
            pltpu.VMEM((1, d), f32),
            pltpu.VMEM((n, s * d), jnp.bfloat16),
            pltpu.VMEM((d, s * d), f32),
            pltpu.VMEM((1, s * d), f32),
            pltpu.VMEM((1, s * d), f32),
            pltpu.VMEM((1, s * d), f32),
            pltpu.VMEM((d, d), f32),
            pltpu.VMEM((1, d), f32),
        ],
        compiler_params=pltpu.CompilerParams(
            dimension_semantics=("arbitrary", "arbitrary"),
        ),
    )(x, W1, b1.reshape(1, dh), W2, b2.reshape(1, s), Ws, bs, gs, betas,
      Wf, bf.reshape(1, d), gf.reshape(1, d), bf2.reshape(1, d))
    return out


# speedup vs baseline: 1.0359x; 1.0359x over previous
import functools

import jax
import jax.numpy as jnp
from jax.experimental import pallas as pl
from jax.experimental.pallas import tpu as pltpu

_EPS = 1e-5


def _fused_kernel(x_ref, w1_ref, b1_ref, w2_ref, b2_ref, ws_ref, bs_ref,
                  gs_ref, betas_ref, wf_ref, bf_ref, gf_ref, bf2_ref,
                  out_ref,
                  csum_ref, zall_ref, wcat_ref, bcat_ref, gcat_ref,
                  betacat_ref, wfc_ref, bfc_ref,
                  *, n_rows, n_blocks, n_strategies, d):
    t = pl.program_id(0)
    i = pl.program_id(1)
    blk = x_ref.shape[0]

    @pl.when(t == 0)
    def _pass0():
        part = jnp.dot(jnp.full((1, blk), 1.0, jnp.float32), x_ref[...],
                       preferred_element_type=jnp.float32)

        @pl.when(i == 0)
        def _():
            csum_ref[...] = part
            for s in range(n_strategies):
                sl = slice(s * d, (s + 1) * d)
                w = ws_ref[s]
                wcat_ref[:, sl] = w - jnp.mean(w, axis=1, keepdims=True)
                b = bs_ref[s:s + 1, :]
                bcat_ref[0:1, sl] = b - jnp.mean(b)
            wf = wf_ref[...]
            wfc_ref[...] = wf - jnp.mean(wf, axis=1, keepdims=True)
            bfc_ref[...] = bf_ref[...] - jnp.mean(bf_ref[...])

        @pl.when(i > 0)
        def _():
            csum_ref[...] = csum_ref[...] + part

        h = (jnp.dot(x_ref[...], wcat_ref[...],
                     preferred_element_type=jnp.float32) + bcat_ref[...])
        inv_d = 1.0 / d
        for s in range(n_strategies):
            hs = h[:, s * d:(s + 1) * d]
            ss = jnp.sum(hs * hs, axis=-1, keepdims=True) * inv_d
            zall_ref[pl.ds(i * blk, blk), s * d:(s + 1) * d] = (
                hs * jax.lax.rsqrt(ss + _EPS))

        @pl.when(i == n_blocks - 1)
        def _():
            gr = csum_ref[...] * (1.0 / n_rows)
            h = jnp.maximum(
                jnp.dot(gr, w1_ref[...], preferred_element_type=jnp.float32)
                + b1_ref[...], 0.0)
            logits = (jnp.dot(h, w2_ref[...],
                              preferred_element_type=jnp.float32)
                      + b2_ref[...])
            m = jnp.max(logits, axis=-1, keepdims=True)
            e = jnp.exp(logits - m)
            sm = e / jnp.sum(e, axis=-1, keepdims=True)
            for s in range(n_strategies):
                sl = slice(s * d, (s + 1) * d)
                sw_s = sm[0:1, s:s + 1]
                gcat_ref[0:1, sl] = gs_ref[s:s + 1, :] * sw_s
                betacat_ref[0:1, sl] = betas_ref[s:s + 1, :] * sw_s

    @pl.when(t == 1)
    def _pass1():
        zb = zall_ref[pl.ds(i * blk, blk), :]
        terms = []
        for s in range(n_strategies):
            sl = slice(s * d, (s + 1) * d)
            terms.append(jnp.maximum(
                zb[:, sl] * gcat_ref[0:1, sl] + betacat_ref[0:1, sl], 0.0))
        while len(terms) > 1:
            terms = [terms[j] + terms[j + 1]
                     for j in range(0, len(terms) - 1, 2)
                     ] + ([terms[-1]] if len(terms) % 2 else [])
        acc = terms[0]
        y = (jnp.dot(acc, wfc_ref[...], preferred_element_type=jnp.float32)
             + bfc_ref[...])
        ss = jnp.sum(y * y, axis=-1, keepdims=True) * (1.0 / d)
        r = jax.lax.rsqrt(ss + _EPS)
        out_ref[...] = jnp.maximum(y * r * gf_ref[...] + bf2_ref[...], 0.0)


def kernel(x, edge_index, W1, b1, W2, b2, Ws, bs, gs, betas, Wf, bf, gf, bf2):
    del edge_index
    n, d = x.shape
    s = Ws.shape[0]
    dh = W1.shape[1]

    blk = 5000
    assert n % blk == 0
    nb = n // blk
    const = lambda t, i: (0, 0)
    f32 = jnp.float32

    out = pl.pallas_call(
        functools.partial(_fused_kernel, n_rows=float(n), n_blocks=nb,
                          n_strategies=s, d=d),
        grid=(2, nb),
        in_specs=[
            pl.BlockSpec((blk, d),
                         lambda t, i: (jnp.where(t == 0, i, nb - 1), 0)),
            pl.BlockSpec((d, dh), const),
            pl.BlockSpec((1, dh), const),
            pl.BlockSpec((dh, s), const),
            pl.BlockSpec((1, s), const),
            pl.BlockSpec((s, d, d), lambda t, i: (0, 0, 0)),
            pl.BlockSpec((s, d), const),
            pl.BlockSpec((s, d), const),
            pl.BlockSpec((s, d), const),
            pl.BlockSpec((d, d), const),
            pl.BlockSpec((1, d), const),
            pl.BlockSpec((1, d), const),
            pl.BlockSpec((1, d), const),
        ],
        out_specs=pl.BlockSpec((blk, d),
                               lambda t, i: (jnp.where(t == 0, 0, i), 0)),
        out_shape=jax.ShapeDtypeStruct((n, d), x.dtype),
        scratch_shapes=[
            pltpu.VMEM((1, d), f32),
            pltpu.VMEM((n, s * d), f32),
            pltpu.VMEM((d, s * d), f32),
            pltpu.VMEM((1, s * d), f32),
            pltpu.VMEM((1, s * d), f32),
            pltpu.VMEM((1, s * d), f32),
            pltpu.VMEM((d, d), f32),
            pltpu.VMEM((1, d), f32),
        ],
        compiler_params=pltpu.CompilerParams(
            dimension_semantics=("arbitrary", "arbitrary"),
        ),
    )(x, W1, b1.reshape(1, dh), W2, b2.reshape(1, s), Ws, bs, gs, betas,
      Wf, bf.reshape(1, d), gf.reshape(1, d), bf2.reshape(1, d))
    return out


# speedup vs baseline: 1.0680x; 1.0310x over previous
import functools

import jax
import jax.numpy as jnp
from jax.experimental import pallas as pl
from jax.experimental.pallas import tpu as pltpu

_EPS = 1e-5


def _fused_kernel(x_ref, w1_ref, b1_ref, w2_ref, b2_ref, ws_ref, bs_ref,
                  gs_ref, betas_ref, wf_ref, bf_ref, gf_ref, bf2_ref,
                  out_ref,
                  csum_ref, zall_ref, wcat_ref, bcat_ref, gcat_ref,
                  betacat_ref, wfc_ref, bfc_ref,
                  *, n_rows, n_blocks, n_strategies, d):
    t = pl.program_id(0)
    i = pl.program_id(1)
    blk = x_ref.shape[0]

    @pl.when(t == 0)
    def _pass0():
        n_chunks = 8
        chunk = blk // n_chunks
        parts = [jnp.sum(x_ref[k * chunk:(k + 1) * chunk, :], axis=0,
                         keepdims=True)
                 for k in range(n_chunks)]
        if blk % n_chunks:
            parts.append(jnp.sum(x_ref[n_chunks * chunk:, :], axis=0,
                                 keepdims=True))
        while len(parts) > 1:
            parts = [parts[j] + parts[j + 1]
                     for j in range(0, len(parts) - 1, 2)
                     ] + ([parts[-1]] if len(parts) % 2 else [])
        part = parts[0]

        @pl.when(i == 0)
        def _():
            csum_ref[...] = part
            for s in range(n_strategies):
                sl = slice(s * d, (s + 1) * d)
                w = ws_ref[s]
                wcat_ref[:, sl] = w - jnp.mean(w, axis=1, keepdims=True)
                b = bs_ref[s:s + 1, :]
                bcat_ref[0:1, sl] = b - jnp.mean(b)
            wf = wf_ref[...]
            wfc_ref[...] = wf - jnp.mean(wf, axis=1, keepdims=True)
            bfc_ref[...] = bf_ref[...] - jnp.mean(bf_ref[...])

        @pl.when(i > 0)
        def _():
            csum_ref[...] = csum_ref[...] + part

        h = (jnp.dot(x_ref[...], wcat_ref[...],
                     preferred_element_type=jnp.float32) + bcat_ref[...])
        inv_d = 1.0 / d
        for s in range(n_strategies):
            hs = h[:, s * d:(s + 1) * d]
            ss = jnp.sum(hs * hs, axis=-1, keepdims=True) * inv_d
            zall_ref[pl.ds(i * blk, blk), s * d:(s + 1) * d] = (
                hs * jax.lax.rsqrt(ss + _EPS))

        @pl.when(i == n_blocks - 1)
        def _():
            gr = csum_ref[...] * (1.0 / n_rows)
            h = jnp.maximum(
                jnp.dot(gr, w1_ref[...], preferred_element_type=jnp.float32)
                + b1_ref[...], 0.0)
            logits = (jnp.dot(h, w2_ref[...],
                              preferred_element_type=jnp.float32)
                      + b2_ref[...])
            m = jnp.max(logits, axis=-1, keepdims=True)
            e = jnp.exp(logits - m)
            sm = e / jnp.sum(e, axis=-1, keepdims=True)
            for s in range(n_strategies):
                sl = slice(s * d, (s + 1) * d)
                sw_s = sm[0:1, s:s + 1]
                gcat_ref[0:1, sl] = gs_ref[s:s + 1, :] * sw_s
                betacat_ref[0:1, sl] = betas_ref[s:s + 1, :] * sw_s

    @pl.when(t == 1)
    def _pass1():
        zb = zall_ref[pl.ds(i * blk, blk), :]
        terms = []
        for s in range(n_strategies):
            sl = slice(s * d, (s + 1) * d)
            terms.append(jnp.maximum(
                zb[:, sl] * gcat_ref[0:1, sl] + betacat_ref[0:1, sl], 0.0))
        while len(terms) > 1:
            terms = [terms[j] + terms[j + 1]
                     for j in range(0, len(terms) - 1, 2)
                     ] + ([terms[-1]] if len(terms) % 2 else [])
        acc = terms[0]
        y = (jnp.dot(acc, wfc_ref[...], preferred_element_type=jnp.float32)
             + bfc_ref[...])
        ss = jnp.sum(y * y, axis=-1, keepdims=True) * (1.0 / d)
        r = jax.lax.rsqrt(ss + _EPS)
        out_ref[...] = jnp.maximum(y * r * gf_ref[...] + bf2_ref[...], 0.0)


def kernel(x, edge_index, W1, b1, W2, b2, Ws, bs, gs, betas, Wf, bf, gf, bf2):
    del edge_index
    n, d = x.shape
    s = Ws.shape[0]
    dh = W1.shape[1]

    blk = 5000
    assert n % blk == 0
    nb = n // blk
    const = lambda t, i: (0, 0)
    f32 = jnp.float32

    out = pl.pallas_call(
        functools.partial(_fused_kernel, n_rows=float(n), n_blocks=nb,
                          n_strategies=s, d=d),
        grid=(2, nb),
        in_specs=[
            pl.BlockSpec((blk, d),
                         lambda t, i: (jnp.where(t == 0, i, nb - 1), 0)),
            pl.BlockSpec((d, dh), const),
            pl.BlockSpec((1, dh), const),
            pl.BlockSpec((dh, s), const),
            pl.BlockSpec((1, s), const),
            pl.BlockSpec((s, d, d), lambda t, i: (0, 0, 0)),
            pl.BlockSpec((s, d), const),
            pl.BlockSpec((s, d), const),
            pl.BlockSpec((s, d), const),
            pl.BlockSpec((d, d), const),
            pl.BlockSpec((1, d), const),
            pl.BlockSpec((1, d), const),
            pl.BlockSpec((1, d), const),
        ],
        out_specs=pl.BlockSpec((blk, d),
                               lambda t, i: (jnp.where(t == 0, 0, i), 0)),
        out_shape=jax.ShapeDtypeStruct((n, d), x.dtype),
        scratch_shapes=[
            pltpu.VMEM((1, d), f32),
            pltpu.VMEM((n, s * d), f32),
            pltpu.VMEM((d, s * d), f32),
            pltpu.VMEM((1, s * d), f32),
            pltpu.VMEM((1, s * d), f32),
            pltpu.VMEM((1, s * d), f32),
            pltpu.VMEM((d, d), f32),
            pltpu.VMEM((1, d), f32),
        ],
        compiler_params=pltpu.CompilerParams(
            dimension_semantics=("arbitrary", "arbitrary"),
        ),
    )(x, W1, b1.reshape(1, dh), W2, b2.reshape(1, s), Ws, bs, gs, betas,
      Wf, bf.reshape(1, d), gf.reshape(1, d), bf2.reshape(1, d))
    return out
